# baseline (speedup 1.0000x reference)
import functools

import jax
import jax.numpy as jnp
from jax import lax
from jax.experimental import pallas as pl
from jax.experimental.pallas import tpu as pltpu
from jax.experimental.pallas import tpu_sc as plsc

N = 4608
K = 8192
D = 256
NB = 256
NUM_SC_WORKERS = 32
BPW = N // NUM_SC_WORKERS
HALF = BPW // 2


KB = 2048
CH = KB // 128


def _argmin_level0_body(x_ref, w_ref, idx_ref, wneg_ref, w2_ref):
    _argmin_common(x_ref[...], w_ref, idx_ref, wneg_ref, w2_ref)


def _argmin_level1_body(x_ref, q_ref, w_ref, idx_ref, wneg_ref, w2_ref):
    _argmin_common(x_ref[...] - q_ref[...], w_ref, idx_ref, wneg_ref, w2_ref)


def _nextup(t):
    return lax.bitcast_convert_type(
        lax.bitcast_convert_type(t, jnp.int32) + 1, jnp.float32)


def _argmin_common(x, w_ref, idx_ref, wneg_ref, w2_ref):
    nb = x.shape[0]

    @pl.when(pl.program_id(0) == 0)
    def _():
        wneg = -2.0 * w_ref[...]
        wneg_ref[...] = wneg
        w2_ref[...] = (jnp.sum(wneg * wneg, axis=1) * 0.25)[None, :]

    x2 = jnp.sum(x * x, axis=1, keepdims=True)

    lm = None
    for kb in range(K // KB):
        wn = wneg_ref[pl.ds(kb * KB, KB), :]
        dotn = lax.dot_general(x, wn, (((1,), (1,)), ((), ())))
        s = x2 + w2_ref[:, kb * KB:(kb + 1) * KB]
        u = s + dotn
        u3 = u.reshape(nb, CH, 128)
        lmb = jnp.min(u3, axis=1)
        cio = lax.broadcasted_iota(jnp.int32, (nb, CH, 128), 1) + (kb * CH)
        cib = jnp.min(jnp.where(u3 == lmb[:, None, :], cio, K // 128), axis=1)
        if lm is None:
            lm, ci = lmb, cib
        else:
            better = (lmb < lm) | ((lmb == lm) & (cib < ci))
            lm = jnp.where(better, lmb, lm)
            ci = jnp.where(better, cib, ci)

    m_u = jnp.min(lm, axis=1, keepdims=True)
    base = jnp.maximum(m_u, 0.0)
    sm = jnp.sqrt(base)
    T = base
    for _ in range(8):
        tn = _nextup(T)
        T = jnp.where(jnp.sqrt(tn) == sm, tn, T)
    lane = lax.broadcasted_iota(jnp.int32, (nb, 128), 1)
    gidx = jnp.where(lm <= T, ci * 128 + lane, K)
    idx = jnp.min(gidx, axis=1)
    idx_ref[...] = jnp.broadcast_to(idx[:, None], (nb, 128))


_x_spec = pl.BlockSpec((NB, D), lambda i: (i, 0))
_w_spec = pl.BlockSpec((K, D), lambda i: (0, 0))
_idx_spec = pl.BlockSpec((NB, 128), lambda i: (i, 0))
_idx_shape = jax.ShapeDtypeStruct((N, 128), jnp.int32)

_argmin_scratch = [
    pltpu.VMEM((K, D), jnp.float32),
    pltpu.VMEM((1, K), jnp.float32),
]

_argmin0 = pl.pallas_call(
    _argmin_level0_body,
    grid=(N // NB,),
    in_specs=[_x_spec, _w_spec],
    out_specs=_idx_spec,
    out_shape=_idx_shape,
    scratch_shapes=_argmin_scratch,
)

_argmin1 = pl.pallas_call(
    _argmin_level1_body,
    grid=(N // NB,),
    in_specs=[_x_spec, _x_spec, _w_spec],
    out_specs=_idx_spec,
    out_shape=_idx_shape,
    scratch_shapes=_argmin_scratch,
)


@functools.cache
def _get_gather_sc():
    @functools.partial(
        pl.kernel,
        mesh=plsc.VectorSubcoreMesh(core_axis_name="c", subcore_axis_name="s"),
        out_type=jax.ShapeDtypeStruct((N, D), jnp.float32),
        scratch_types=[
            pltpu.VMEM((BPW,), jnp.int32),
            pltpu.VMEM((BPW, D), jnp.float32),
            pltpu.SemaphoreType.DMA,
        ],
    )
    def _gather_sc(table_hbm, idx_hbm, out_hbm, idx_v, rows_v, sem):
        wid = lax.axis_index("s") * 2 + lax.axis_index("c")
        base = wid * BPW
        pltpu.sync_copy(idx_hbm.at[pl.ds(base, BPW)], idx_v)
        cp0 = pltpu.async_copy(
            table_hbm.at[idx_v.at[pl.ds(0, HALF)]], rows_v.at[pl.ds(0, HALF)], sem)
        cp1 = pltpu.async_copy(
            table_hbm.at[idx_v.at[pl.ds(HALF, HALF)]], rows_v.at[pl.ds(HALF, HALF)], sem)
        cp0.wait()
        cp1.wait()
        pltpu.sync_copy(rows_v, out_hbm.at[pl.ds(base, BPW)])

    return _gather_sc


def _final_body(x_ref, q0_ref, q1_ref, out_ref, s0_ref, s1_ref):
    i = pl.program_id(0)
    x = x_ref[...]
    q0 = q0_ref[...]
    q1 = q1_ref[...]
    code_sum = q0 + q1
    out_ref[...] = x + (code_sum - x)
    d0 = q0 - x
    d1 = q1 - (x - q0)

    @pl.when(i == 0)
    def _():
        s0_ref[0, 0] = 0.0
        s1_ref[0, 0] = 0.0

    s0_ref[0, 0] += jnp.sum(d0 * d0)
    s1_ref[0, 0] += jnp.sum(d1 * d1)


_final = pl.pallas_call(
    _final_body,
    grid=(N // NB,),
    in_specs=[_x_spec, _x_spec, _x_spec],
    out_specs=[
        pl.BlockSpec((NB, D), lambda i: (i, 0)),
        pl.BlockSpec(memory_space=pltpu.SMEM, block_shape=(1, 1), index_map=lambda i: (0, 0)),
        pl.BlockSpec(memory_space=pltpu.SMEM, block_shape=(1, 1), index_map=lambda i: (0, 0)),
    ],
    out_shape=[
        jax.ShapeDtypeStruct((N, D), jnp.float32),
        jax.ShapeDtypeStruct((1, 1), jnp.float32),
        jax.ShapeDtypeStruct((1, 1), jnp.float32),
    ],
)


def kernel(latent, W0, W1):
    gather_sc = _get_gather_sc()
    idx0 = _argmin0(latent, W0)[:, 0]
    q0 = gather_sc(W0, idx0)
    idx1 = _argmin1(latent, q0, W1)[:, 0]
    q1 = gather_sc(W1, idx1)
    out, s0, s1 = _final(latent, q0, q1)
    nd = jnp.float32(N * D)
    l0 = s0[0, 0] / nd
    l1 = s1[0, 0] / nd
    loss = l0 + 0.25 * l0 + l1 + 0.25 * l1
    return (loss, out)


# speedup vs baseline: 1.7613x; 1.7613x over previous
import functools

import jax
import jax.numpy as jnp
from jax import lax
from jax.experimental import pallas as pl
from jax.experimental.pallas import tpu as pltpu
from jax.experimental.pallas import tpu_sc as plsc

N = 4608
K = 8192
D = 256
NB = 256
NUM_SC_WORKERS = 32
BPW = N // NUM_SC_WORKERS
HALF = BPW // 2


KB = 2048
CH = KB // 128


def _argmin_level0_body(x_ref, w_ref, idx_ref, wneg_ref, w2_ref, u_ref):
    _argmin_common(x_ref[...], w_ref, idx_ref, wneg_ref, w2_ref, u_ref)


def _argmin_level1_body(x_ref, q_ref, w_ref, idx_ref, wneg_ref, w2_ref, u_ref):
    _argmin_common(x_ref[...] - q_ref[...], w_ref, idx_ref, wneg_ref, w2_ref, u_ref)


def _nextup(t):
    return lax.bitcast_convert_type(
        lax.bitcast_convert_type(t, jnp.int32) + 1, jnp.float32)


def _argmin_common(x, w_ref, idx_ref, wneg_ref, w2_ref, u_ref):
    nb = x.shape[0]

    @pl.when(pl.program_id(0) == 0)
    def _():
        wneg = -2.0 * w_ref[...]
        wneg_ref[...] = wneg
        w2_ref[...] = (jnp.sum(wneg * wneg, axis=1) * 0.25)[None, :]

    x2 = jnp.sum(x * x, axis=1, keepdims=True)

    m_u = None
    for kb in range(K // KB):
        wn = wneg_ref[pl.ds(kb * KB, KB), :]
        dotn = lax.dot_general(x, wn, (((1,), (1,)), ((), ())))
        u = (x2 + w2_ref[:, kb * KB:(kb + 1) * KB]) + dotn
        u_ref[:, kb * KB:(kb + 1) * KB] = u
        mb = jnp.min(u, axis=1, keepdims=True)
        m_u = mb if m_u is None else jnp.minimum(m_u, mb)

    base = jnp.maximum(m_u, 0.0)
    sm = jnp.sqrt(base)
    T = base
    for _ in range(8):
        tn = _nextup(T)
        T = jnp.where(jnp.sqrt(tn) == sm, tn, T)

    u = u_ref[...]
    io = lax.broadcasted_iota(jnp.int32, (nb, K), 1)
    idx = jnp.min(jnp.where(u <= T, io, K), axis=1)
    idx_ref[...] = jnp.broadcast_to(idx[:, None], (nb, 128))


_x_spec = pl.BlockSpec((NB, D), lambda i: (i, 0))
_w_spec = pl.BlockSpec((K, D), lambda i: (0, 0))
_idx_spec = pl.BlockSpec((NB, 128), lambda i: (i, 0))
_idx_shape = jax.ShapeDtypeStruct((N, 128), jnp.int32)

_argmin_scratch = [
    pltpu.VMEM((K, D), jnp.float32),
    pltpu.VMEM((1, K), jnp.float32),
    pltpu.VMEM((NB, K), jnp.float32),
]

_argmin0 = pl.pallas_call(
    _argmin_level0_body,
    grid=(N // NB,),
    in_specs=[_x_spec, _w_spec],
    out_specs=_idx_spec,
    out_shape=_idx_shape,
    scratch_shapes=_argmin_scratch,
)

_argmin1 = pl.pallas_call(
    _argmin_level1_body,
    grid=(N // NB,),
    in_specs=[_x_spec, _x_spec, _w_spec],
    out_specs=_idx_spec,
    out_shape=_idx_shape,
    scratch_shapes=_argmin_scratch,
)


@functools.cache
def _get_gather_sc():
    @functools.partial(
        pl.kernel,
        mesh=plsc.VectorSubcoreMesh(core_axis_name="c", subcore_axis_name="s"),
        out_type=jax.ShapeDtypeStruct((N, D), jnp.float32),
        scratch_types=[
            pltpu.VMEM((BPW,), jnp.int32),
            pltpu.VMEM((BPW, D), jnp.float32),
            pltpu.SemaphoreType.DMA,
        ],
    )
    def _gather_sc(table_hbm, idx_hbm, out_hbm, idx_v, rows_v, sem):
        wid = lax.axis_index("s") * 2 + lax.axis_index("c")
        base = wid * BPW
        pltpu.sync_copy(idx_hbm.at[pl.ds(base, BPW)], idx_v)
        cp0 = pltpu.async_copy(
            table_hbm.at[idx_v.at[pl.ds(0, HALF)]], rows_v.at[pl.ds(0, HALF)], sem)
        cp1 = pltpu.async_copy(
            table_hbm.at[idx_v.at[pl.ds(HALF, HALF)]], rows_v.at[pl.ds(HALF, HALF)], sem)
        cp0.wait()
        cp1.wait()
        pltpu.sync_copy(rows_v, out_hbm.at[pl.ds(base, BPW)])

    return _gather_sc


def _final_body(x_ref, q0_ref, q1_ref, out_ref, s0_ref, s1_ref):
    i = pl.program_id(0)
    x = x_ref[...]
    q0 = q0_ref[...]
    q1 = q1_ref[...]
    code_sum = q0 + q1
    out_ref[...] = x + (code_sum - x)
    d0 = q0 - x
    d1 = q1 - (x - q0)

    @pl.when(i == 0)
    def _():
        s0_ref[0, 0] = 0.0
        s1_ref[0, 0] = 0.0

    s0_ref[0, 0] += jnp.sum(d0 * d0)
    s1_ref[0, 0] += jnp.sum(d1 * d1)


_final = pl.pallas_call(
    _final_body,
    grid=(N // NB,),
    in_specs=[_x_spec, _x_spec, _x_spec],
    out_specs=[
        pl.BlockSpec((NB, D), lambda i: (i, 0)),
        pl.BlockSpec(memory_space=pltpu.SMEM, block_shape=(1, 1), index_map=lambda i: (0, 0)),
        pl.BlockSpec(memory_space=pltpu.SMEM, block_shape=(1, 1), index_map=lambda i: (0, 0)),
    ],
    out_shape=[
        jax.ShapeDtypeStruct((N, D), jnp.float32),
        jax.ShapeDtypeStruct((1, 1), jnp.float32),
        jax.ShapeDtypeStruct((1, 1), jnp.float32),
    ],
)


def kernel(latent, W0, W1):
    gather_sc = _get_gather_sc()
    idx0 = _argmin0(latent, W0)[:, 0]
    q0 = gather_sc(W0, idx0)
    idx1 = _argmin1(latent, q0, W1)[:, 0]
    q1 = gather_sc(W1, idx1)
    out, s0, s1 = _final(latent, q0, q1)
    nd = jnp.float32(N * D)
    l0 = s0[0, 0] / nd
    l1 = s1[0, 0] / nd
    loss = l0 + 0.25 * l0 + l1 + 0.25 * l1
    return (loss, out)


# speedup vs baseline: 1.9551x; 1.1100x over previous
import functools

import jax
import jax.numpy as jnp
from jax import lax
from jax.experimental import pallas as pl
from jax.experimental.pallas import tpu as pltpu
from jax.experimental.pallas import tpu_sc as plsc

N = 4608
K = 8192
D = 256
NB = 384
_IDX_BIAS_BITS = 0x4B000000
_IDX_BIAS = 8388608.0
NUM_SC_WORKERS = 32
BPW = N // NUM_SC_WORKERS
HALF = BPW // 2


KB = 2048
CH = KB // 128


def _argmin_level0_body(x_ref, w_ref, idx_ref, wneg_ref, w2_ref, u_ref, io_ref):
    _argmin_common(x_ref[...], w_ref, idx_ref, wneg_ref, w2_ref, u_ref, io_ref)


def _argmin_level1_body(x_ref, q_ref, w_ref, idx_ref, wneg_ref, w2_ref, u_ref,
                        io_ref):
    _argmin_common(x_ref[...] - q_ref[...], w_ref, idx_ref, wneg_ref, w2_ref,
                   u_ref, io_ref)


def _nextup(t):
    return lax.bitcast_convert_type(
        lax.bitcast_convert_type(t, jnp.int32) + 1, jnp.float32)


def _argmin_common(x, w_ref, idx_ref, wneg_ref, w2_ref, u_ref, io_ref):
    nb = x.shape[0]

    @pl.when(pl.program_id(0) == 0)
    def _():
        wneg = -2.0 * w_ref[...]
        wneg_ref[...] = wneg
        w2_ref[...] = (jnp.sum(wneg * wneg, axis=1) * 0.25)[None, :]
        io_ref[...] = lax.bitcast_convert_type(
            lax.broadcasted_iota(jnp.int32, (nb, K), 1) + _IDX_BIAS_BITS,
            jnp.float32)

    x2 = jnp.sum(x * x, axis=1, keepdims=True)

    m_u = None
    for kb in range(K // KB):
        wn = wneg_ref[pl.ds(kb * KB, KB), :]
        dotn = lax.dot_general(x, wn, (((1,), (1,)), ((), ())))
        u = (x2 + w2_ref[:, kb * KB:(kb + 1) * KB]) + dotn
        u_ref[:, kb * KB:(kb + 1) * KB] = u
        mb = jnp.min(u, axis=1, keepdims=True)
        m_u = mb if m_u is None else jnp.minimum(m_u, mb)

    base = jnp.maximum(m_u, 0.0)
    sm = jnp.sqrt(base)
    T = base
    for _ in range(8):
        tn = _nextup(T)
        T = jnp.where(jnp.sqrt(tn) == sm, tn, T)

    u = u_ref[...]
    iof = io_ref[...]
    idx_f = jnp.min(jnp.where(u <= T, iof, jnp.float32(2.0 * _IDX_BIAS)), axis=1)
    idx = (idx_f - jnp.float32(_IDX_BIAS)).astype(jnp.int32)
    idx_ref[...] = jnp.broadcast_to(idx[:, None], (nb, 128))


_x_spec = pl.BlockSpec((NB, D), lambda i: (i, 0))
_w_spec = pl.BlockSpec((K, D), lambda i: (0, 0))
_idx_spec = pl.BlockSpec((NB, 128), lambda i: (i, 0))
_idx_shape = jax.ShapeDtypeStruct((N, 128), jnp.int32)

_argmin_scratch = [
    pltpu.VMEM((K, D), jnp.float32),
    pltpu.VMEM((1, K), jnp.float32),
    pltpu.VMEM((NB, K), jnp.float32),
    pltpu.VMEM((NB, K), jnp.float32),
]

_argmin0 = pl.pallas_call(
    _argmin_level0_body,
    grid=(N // NB,),
    in_specs=[_x_spec, _w_spec],
    out_specs=_idx_spec,
    out_shape=_idx_shape,
    scratch_shapes=_argmin_scratch,
)

_argmin1 = pl.pallas_call(
    _argmin_level1_body,
    grid=(N // NB,),
    in_specs=[_x_spec, _x_spec, _w_spec],
    out_specs=_idx_spec,
    out_shape=_idx_shape,
    scratch_shapes=_argmin_scratch,
)


@functools.cache
def _get_gather_sc():
    @functools.partial(
        pl.kernel,
        mesh=plsc.VectorSubcoreMesh(core_axis_name="c", subcore_axis_name="s"),
        out_type=jax.ShapeDtypeStruct((N, D), jnp.float32),
        scratch_types=[
            pltpu.VMEM((BPW,), jnp.int32),
            pltpu.VMEM((BPW, D), jnp.float32),
            pltpu.SemaphoreType.DMA,
        ],
    )
    def _gather_sc(table_hbm, idx_hbm, out_hbm, idx_v, rows_v, sem):
        wid = lax.axis_index("s") * 2 + lax.axis_index("c")
        base = wid * BPW
        pltpu.sync_copy(idx_hbm.at[pl.ds(base, BPW)], idx_v)
        cp0 = pltpu.async_copy(
            table_hbm.at[idx_v.at[pl.ds(0, HALF)]], rows_v.at[pl.ds(0, HALF)], sem)
        cp1 = pltpu.async_copy(
            table_hbm.at[idx_v.at[pl.ds(HALF, HALF)]], rows_v.at[pl.ds(HALF, HALF)], sem)
        cp0.wait()
        cp1.wait()
        pltpu.sync_copy(rows_v, out_hbm.at[pl.ds(base, BPW)])

    return _gather_sc


def _final_body(x_ref, q0_ref, q1_ref, out_ref, s0_ref, s1_ref):
    i = pl.program_id(0)
    x = x_ref[...]
    q0 = q0_ref[...]
    q1 = q1_ref[...]
    code_sum = q0 + q1
    out_ref[...] = x + (code_sum - x)
    d0 = q0 - x
    d1 = q1 - (x - q0)

    @pl.when(i == 0)
    def _():
        s0_ref[0, 0] = 0.0
        s1_ref[0, 0] = 0.0

    s0_ref[0, 0] += jnp.sum(d0 * d0)
    s1_ref[0, 0] += jnp.sum(d1 * d1)


_final = pl.pallas_call(
    _final_body,
    grid=(N // NB,),
    in_specs=[_x_spec, _x_spec, _x_spec],
    out_specs=[
        pl.BlockSpec((NB, D), lambda i: (i, 0)),
        pl.BlockSpec(memory_space=pltpu.SMEM, block_shape=(1, 1), index_map=lambda i: (0, 0)),
        pl.BlockSpec(memory_space=pltpu.SMEM, block_shape=(1, 1), index_map=lambda i: (0, 0)),
    ],
    out_shape=[
        jax.ShapeDtypeStruct((N, D), jnp.float32),
        jax.ShapeDtypeStruct((1, 1), jnp.float32),
        jax.ShapeDtypeStruct((1, 1), jnp.float32),
    ],
)


def kernel(latent, W0, W1):
    gather_sc = _get_gather_sc()
    idx0 = _argmin0(latent, W0)[:, 0]
    q0 = gather_sc(W0, idx0)
    idx1 = _argmin1(latent, q0, W1)[:, 0]
    q1 = gather_sc(W1, idx1)
    out, s0, s1 = _final(latent, q0, q1)
    nd = jnp.float32(N * D)
    l0 = s0[0, 0] / nd
    l1 = s1[0, 0] / nd
    loss = l0 + 0.25 * l0 + l1 + 0.25 * l1
    return (loss, out)
